# baseline (speedup 1.0000x reference)
import functools

import jax
import jax.numpy as jnp
from jax import lax
from jax.experimental import pallas as pl
from jax.experimental.pallas import tpu as pltpu
from jax.experimental.pallas import tpu_sc as plsc

D = 64
T = 200
B = 4096
NC, NS = 2, 16
NW = NC * NS
ROWS_PER_W = B // NW
HALF = T // 2

_mesh = plsc.VectorSubcoreMesh(core_axis_name="c", subcore_axis_name="s")


@functools.partial(
    pl.kernel,
    out_type=jax.ShapeDtypeStruct((B, T, D), jnp.float32),
    mesh=_mesh,
    scratch_types=[
        pltpu.VMEM((2, HALF), jnp.int32),
        pltpu.VMEM((T, D), jnp.float32),
        pltpu.VMEM((T, D), jnp.float32),
        pltpu.SemaphoreType.DMA,
    ],
    compiler_params=pltpu.CompilerParams(use_tc_tiling_on_sc=False),
)
def _embed(x_hbm, tok_hbm, pos_hbm, out_hbm, idx_v, buf_v, pos_v, sem):
    wid = lax.axis_index("s") * NC + lax.axis_index("c")
    pltpu.sync_copy(pos_hbm, pos_v)

    @pl.loop(0, ROWS_PER_W)
    def _(i):
        b = wid * ROWS_PER_W + i
        pltpu.sync_copy(x_hbm.at[b], idx_v)
        cp0 = pltpu.async_copy(
            tok_hbm.at[idx_v.at[0]], buf_v.at[pl.ds(0, HALF)], sem)
        cp1 = pltpu.async_copy(
            tok_hbm.at[idx_v.at[1]], buf_v.at[pl.ds(HALF, HALF)], sem)
        cp0.wait()
        cp1.wait()

        @pl.loop(0, T)
        def _(r):
            for c in range(D // 16):
                sl = pl.ds(c * 16, 16)
                buf_v[r, sl] += pos_v[r, sl]

        pltpu.sync_copy(buf_v, out_hbm.at[b])


def kernel(x, token_table, pos_table):
    x32 = x.astype(jnp.int32).reshape(B, 2, HALF)
    return _embed(x32, token_table, pos_table)
